# baseline (speedup 1.0000x reference)
import functools

import jax
import jax.numpy as jnp
from jax import lax
from jax.experimental import pallas as pl
from jax.experimental.pallas import tpu as pltpu
from jax.experimental.pallas import tpu_sc as plsc

VOCAB = 1000
D = 4
DP = 16
NC = 2
NS = 16
NW = NC * NS
CHUNK = 128
TP = 56
CH = 512
BB2 = 32
PB = CH // NW


def _sc_gather(table_p, ids3, btot):
    n_chunks = ids3.shape[1]
    per_w = n_chunks * CHUNK
    mesh = plsc.VectorSubcoreMesh(core_axis_name="c", subcore_axis_name="s")

    @functools.partial(
        pl.kernel,
        mesh=mesh,
        out_type=jax.ShapeDtypeStruct((btot, DP), jnp.float32),
        scratch_types=[
            pltpu.VMEM((n_chunks, CHUNK), jnp.int32),
            pltpu.VMEM((per_w, DP), jnp.float32),
            pltpu.SemaphoreType.DMA,
        ],
        compiler_params=pltpu.CompilerParams(use_tc_tiling_on_sc=False),
    )
    def k(table_hbm, ids_hbm, out_hbm, idx_v, rows_v, sem):
        wid = lax.axis_index("s") * NC + lax.axis_index("c")
        pltpu.sync_copy(ids_hbm.at[wid], idx_v)

        def body(j, carry):
            pltpu.async_copy(
                table_hbm.at[idx_v.at[j]], rows_v.at[pl.ds(j * CHUNK, CHUNK)], sem
            ).wait()
            return carry

        lax.fori_loop(0, n_chunks, body, 0)
        pltpu.sync_copy(rows_v, out_hbm.at[pl.ds(wid * per_w, per_w)])

    return k(table_p, ids3)


def _tc_mm_chunk(h, head_t, k, t):
    grid_n = CH // BB2

    def body(h_ref, w_ref, o_ref):
        w = w_ref[...]
        for bb in range(BB2):
            o_ref[pl.ds(bb * TP, t), :] = lax.dot_general(
                h_ref[pl.ds(bb * t, t), :], w,
                (((1,), (0,)), ((), ())),
                preferred_element_type=jnp.float32,
            )

    return pl.pallas_call(
        body,
        grid=(grid_n,),
        in_specs=[
            pl.BlockSpec((BB2 * t, DP), lambda i, k=k: (k * grid_n + i, 0)),
            pl.BlockSpec((DP, VOCAB), lambda i: (0, 0)),
        ],
        out_specs=pl.BlockSpec((BB2 * TP, VOCAB), lambda i: (i, 0)),
        out_shape=jax.ShapeDtypeStruct((CH * TP, VOCAB), jnp.float32),
    )(h, head_t)


def _sc_repack(mm_k, buf_ref, k, t):
    mesh = plsc.VectorSubcoreMesh(core_axis_name="c", subcore_axis_name="s")

    @functools.partial(
        pl.kernel,
        mesh=mesh,
        out_type=(),
        scratch_types=[pltpu.VMEM((TP, VOCAB), jnp.float32)],
        compiler_params=pltpu.CompilerParams(use_tc_tiling_on_sc=True),
    )
    def rk(mm_hbm, buf_hbm, tmp_v):
        wid = lax.axis_index("s") * NC + lax.axis_index("c")

        def body(i, carry):
            b_local = wid * PB + i
            gb = k * CH + b_local
            pltpu.sync_copy(mm_hbm.at[pl.ds(b_local * TP, TP)], tmp_v)
            pltpu.sync_copy(tmp_v.at[pl.ds(0, 48)], buf_hbm.at[gb, pl.ds(0, 48), :])
            pltpu.sync_copy(tmp_v.at[48], buf_hbm.at[gb, 48, :])
            pltpu.sync_copy(tmp_v.at[49], buf_hbm.at[gb, 49, :])
            return carry

        lax.fori_loop(0, PB, body, 0)

    rk(mm_k, buf_ref)


def kernel(input_ids, emb_weight, head_weight):
    b, t = input_ids.shape
    btot = b * t
    n_chunks_total = b // CH
    ids3 = input_ids.astype(jnp.int32).reshape(NW, btot // (NW * CHUNK), CHUNK)
    emb_p = jnp.pad(emb_weight, ((0, 0), (0, DP - D)))
    head_t = jnp.pad(head_weight, ((0, 0), (0, DP - D))).T
    h = _sc_gather(emb_p, ids3, btot)
    buf = jax.ref.empty_ref(jax.ShapeDtypeStruct((b, t, VOCAB), jnp.float32))
    for k in range(n_chunks_total):
        mm_k = _tc_mm_chunk(h, head_t, k, t)
        _sc_repack(mm_k, buf, k, t)
    return jax.ref.freeze(buf)


# speedup vs baseline: 1.2376x; 1.2376x over previous
import functools

import jax
import jax.numpy as jnp
from jax import lax
from jax.experimental import pallas as pl
from jax.experimental.pallas import tpu as pltpu
from jax.experimental.pallas import tpu_sc as plsc

VOCAB = 1000
D = 4
DP = 16
NC = 2
NS = 16
NW = NC * NS
CHUNK = 128
TP = 56
CH = 512
BB2 = 32
PB = CH // NW


def _sc_gather(table_p, ids3, btot):
    n_chunks = ids3.shape[1]
    per_w = n_chunks * CHUNK
    mesh = plsc.VectorSubcoreMesh(core_axis_name="c", subcore_axis_name="s")

    @functools.partial(
        pl.kernel,
        mesh=mesh,
        out_type=jax.ShapeDtypeStruct((btot, DP), jnp.float32),
        scratch_types=[
            pltpu.VMEM((n_chunks, CHUNK), jnp.int32),
            pltpu.VMEM((per_w, DP), jnp.float32),
            pltpu.SemaphoreType.DMA,
        ],
        compiler_params=pltpu.CompilerParams(use_tc_tiling_on_sc=False),
    )
    def k(table_hbm, ids_hbm, out_hbm, idx_v, rows_v, sem):
        wid = lax.axis_index("s") * NC + lax.axis_index("c")
        pltpu.sync_copy(ids_hbm.at[wid], idx_v)

        def body(j, carry):
            pltpu.async_copy(
                table_hbm.at[idx_v.at[j]], rows_v.at[pl.ds(j * CHUNK, CHUNK)], sem
            ).wait()
            return carry

        lax.fori_loop(0, n_chunks, body, 0)
        pltpu.sync_copy(rows_v, out_hbm.at[pl.ds(wid * per_w, per_w)])

    return k(table_p, ids3)


VP = 1024


def _tc_mm_chunk(h, head_t, k, t):
    grid_n = CH // BB2

    def body(h_ref, w_ref, o_ref):
        w = w_ref[...]
        for bb in range(BB2):
            o_ref[bb, :t, :] = lax.dot_general(
                h_ref[pl.ds(bb * t, t), :], w,
                (((1,), (0,)), ((), ())),
                preferred_element_type=jnp.float32,
            )

    return pl.pallas_call(
        body,
        grid=(grid_n,),
        in_specs=[
            pl.BlockSpec((BB2 * t, DP), lambda i, k=k: (k * grid_n + i, 0)),
            pl.BlockSpec((DP, VP), lambda i: (0, 0)),
        ],
        out_specs=pl.BlockSpec((BB2, TP, VP), lambda i: (i, 0, 0)),
        out_shape=jax.ShapeDtypeStruct((CH, TP, VP), jnp.float32),
    )(h, head_t)


def kernel(input_ids, emb_weight, head_weight):
    b, t = input_ids.shape
    btot = b * t
    n_chunks_total = b // CH
    ids3 = input_ids.astype(jnp.int32).reshape(NW, btot // (NW * CHUNK), CHUNK)
    emb_p = jnp.pad(emb_weight, ((0, 0), (0, DP - D)))
    head_t = jnp.pad(jnp.pad(head_weight, ((0, 0), (0, DP - D))).T,
                     ((0, 0), (0, VP - VOCAB)))
    h = _sc_gather(emb_p, ids3, btot)
    buf = jax.ref.empty_ref(jax.ShapeDtypeStruct((b, t, VOCAB), jnp.float32))
    for k in range(n_chunks_total):
        pk = _tc_mm_chunk(h, head_t, k, t)
        buf[k * CH:(k + 1) * CH] = pk[:, :t, :VOCAB]
    return jax.ref.freeze(buf)


# speedup vs baseline: 1.8853x; 1.5234x over previous
import functools

import jax
import jax.numpy as jnp
from jax import lax
from jax.experimental import pallas as pl
from jax.experimental.pallas import tpu as pltpu
from jax.experimental.pallas import tpu_sc as plsc

VOCAB = 1000
D = 4
DP = 16
NC = 2
NS = 16
NW = NC * NS
CHUNK = 128
TP = 56
CH = 4096
BB2 = 32


def _sc_gather(table_p, ids3, btot):
    n_chunks = ids3.shape[1]
    per_w = n_chunks * CHUNK
    mesh = plsc.VectorSubcoreMesh(core_axis_name="c", subcore_axis_name="s")

    @functools.partial(
        pl.kernel,
        mesh=mesh,
        out_type=jax.ShapeDtypeStruct((btot, DP), jnp.float32),
        scratch_types=[
            pltpu.VMEM((n_chunks, CHUNK), jnp.int32),
            pltpu.VMEM((per_w, DP), jnp.float32),
            pltpu.SemaphoreType.DMA,
        ],
        compiler_params=pltpu.CompilerParams(use_tc_tiling_on_sc=False),
    )
    def k(table_hbm, ids_hbm, out_hbm, idx_v, rows_v, sem):
        wid = lax.axis_index("s") * NC + lax.axis_index("c")
        pltpu.sync_copy(ids_hbm.at[wid], idx_v)

        def body(j, carry):
            pltpu.async_copy(
                table_hbm.at[idx_v.at[j]], rows_v.at[pl.ds(j * CHUNK, CHUNK)], sem
            ).wait()
            return carry

        lax.fori_loop(0, n_chunks, body, 0)
        pltpu.sync_copy(rows_v, out_hbm.at[pl.ds(wid * per_w, per_w)])

    return k(table_p, ids3)


VP = 1024


def _tc_mm_chunk(h, head_t, k, t):
    grid_n = CH // BB2

    def body(h_ref, w_ref, o_ref):
        w = w_ref[...]
        for bb in range(BB2):
            o_ref[bb, :t, :] = lax.dot_general(
                h_ref[pl.ds(bb * t, t), :], w,
                (((1,), (0,)), ((), ())),
                preferred_element_type=jnp.float32,
            )

    return pl.pallas_call(
        body,
        grid=(grid_n,),
        in_specs=[
            pl.BlockSpec((BB2 * t, DP), lambda i, k=k: (k * grid_n + i, 0)),
            pl.BlockSpec((DP, VP), lambda i: (0, 0)),
        ],
        out_specs=pl.BlockSpec((BB2, TP, VP), lambda i: (i, 0, 0)),
        out_shape=jax.ShapeDtypeStruct((CH, TP, VP), jnp.float32),
    )(h, head_t)


def kernel(input_ids, emb_weight, head_weight):
    b, t = input_ids.shape
    btot = b * t
    ids3 = input_ids.astype(jnp.int32).reshape(NW, btot // (NW * CHUNK), CHUNK)
    emb_p = jnp.pad(emb_weight, ((0, 0), (0, DP - D)))
    head_t = jnp.pad(jnp.pad(head_weight, ((0, 0), (0, DP - D))).T,
                     ((0, 0), (0, VP - VOCAB)))
    h = _sc_gather(emb_p, ids3, btot)
    pk = _tc_mm_chunk(h, head_t, 0, t)
    return pk[:, :t, :VOCAB]
